# Initial kernel scaffold; baseline (speedup 1.0000x reference)
import jax
import jax.numpy as jnp
from jax.experimental import pallas as pl


def kernel(x, ln_gamma, ln_beta, W_router, W_experts):
    raise NotImplementedError("write your pallas kernel here")


# speedup vs baseline: 3.1206x; 3.1206x over previous
import functools

import jax
import jax.numpy as jnp
from jax import lax
from jax.experimental import pallas as pl
from jax.experimental.pallas import tpu as pltpu

EPS = 1e-5
SILU_SCALE = 1.0 / 0.6


def _moe_block_kernel(x_ref, g_ref, b_ref, wr_ref, we_ref, o_ref, *, n_experts):
    x = x_ref[...]
    mu = jnp.mean(x, axis=-1, keepdims=True)
    xc = x - mu
    var = jnp.mean(xc * xc, axis=-1, keepdims=True)
    xn = xc * lax.rsqrt(var + EPS)
    xn = xn * g_ref[...] + b_ref[...]

    logits = lax.dot_general(xn, wr_ref[...], (((1,), (1,)), ((), ())),
                             preferred_element_type=jnp.float32)
    e_iota = lax.broadcasted_iota(jnp.int32, logits.shape, 1)
    neg = jnp.float32(-jnp.inf)
    big = jnp.int32(n_experts)

    m1 = jnp.max(logits, axis=1, keepdims=True)
    is1 = logits == m1
    a1 = jnp.min(jnp.where(is1, e_iota, big), axis=1, keepdims=True)
    first1 = e_iota == a1
    l_rest = jnp.where(first1, neg, logits)
    m2 = jnp.max(l_rest, axis=1, keepdims=True)
    is2 = l_rest == m2
    a2 = jnp.min(jnp.where(is2, e_iota, big), axis=1, keepdims=True)
    first2 = e_iota == a2

    w1 = 1.0 / (1.0 + jnp.exp(m2 - m1))
    gates = jnp.where(first1, w1, 0.0) + jnp.where(first2, 1.0 - w1, 0.0)

    acc = jnp.zeros(o_ref.shape, dtype=jnp.float32)
    for e in range(n_experts):
        y = lax.dot_general(xn, we_ref[e], (((1,), (1,)), ((), ())),
                            preferred_element_type=jnp.float32)
        acc = acc + gates[:, e:e + 1] * y
    o_ref[...] = acc * (1.0 / (1.0 + jnp.exp(-acc))) * SILU_SCALE


def kernel(x, ln_gamma, ln_beta, W_router, W_experts):
    n, d_in = x.shape
    n_experts, d_out, _ = W_experts.shape
    block = 512 if n % 512 == 0 else n
    grid = (n // block,)
    fn = functools.partial(_moe_block_kernel, n_experts=n_experts)
    return pl.pallas_call(
        fn,
        grid=grid,
        in_specs=[
            pl.BlockSpec((block, d_in), lambda t: (t, 0)),
            pl.BlockSpec((1, d_in), lambda t: (0, 0)),
            pl.BlockSpec((1, d_in), lambda t: (0, 0)),
            pl.BlockSpec((n_experts, d_in), lambda t: (0, 0)),
            pl.BlockSpec((n_experts, d_out, d_in), lambda t: (0, 0, 0)),
        ],
        out_specs=pl.BlockSpec((block, d_out), lambda t: (t, 0)),
        out_shape=jax.ShapeDtypeStruct((n, d_out), jnp.float32),
        compiler_params=pltpu.CompilerParams(
            dimension_semantics=("arbitrary",),
        ),
    )(x, ln_gamma.reshape(1, d_in), ln_beta.reshape(1, d_in),
      W_router, W_experts)
